# baseline (speedup 1.0000x reference)
import functools

import jax
import jax.numpy as jnp
from jax import lax
from jax.experimental import pallas as pl
from jax.experimental.pallas import tpu as pltpu
from jax.experimental.pallas import tpu_sc as plsc

NC = 2
NS = 16
NW = NC * NS
CHUNK = 64
SLOW_C = 0
FAST_FRAC = 1.0


def _sc_mesh():
    return plsc.VectorSubcoreMesh(
        core_axis_name="c", subcore_axis_name="s", num_cores=NC,
        num_subcores=NS)


def _fill_f32(ref, n_rows, n_cols, value):
    def row(j, _):
        def col(k, _):
            ref[j, pl.ds(k * 16, 16)] = jnp.full((16,), value, jnp.float32)
            return _
        return lax.fori_loop(0, n_cols // 16, col, _)
    lax.fori_loop(0, n_rows, row, None)


def _deg_body(ec, n_pad, dst_hbm, deg_hbm, dst_v, ones_v, zrow_v, deg_sh):
    c = lax.axis_index("c")
    s = lax.axis_index("s")
    wid = s * NC + c
    rpt = n_pad // NS

    _fill_f32(ones_v, 1, CHUNK, 1.0)
    _fill_f32(zrow_v, 1, rpt, 0.0)
    pltpu.sync_copy(zrow_v.at[0], deg_sh.at[pl.ds(s * rpt, rpt)])
    plsc.subcore_barrier()

    pltpu.sync_copy(dst_hbm.at[pl.ds(wid * ec, ec)], dst_v)

    def edge(j, _):
        pltpu.sync_copy(ones_v.at[0], deg_sh.at[dst_v.at[j]], add=True)
        return _
    lax.fori_loop(0, ec, edge, None)
    plsc.subcore_barrier()

    pltpu.sync_copy(deg_sh.at[pl.ds(s * rpt, rpt)],
                    deg_hbm.at[pl.ds(c * n_pad + s * rpt, rpt)])


def _sc_degree(dst_p, n_pad):
    ec = dst_p.shape[0] // NW
    kern = pl.kernel(
        functools.partial(_deg_body, ec, n_pad),
        out_type=jax.ShapeDtypeStruct((NC * n_pad,), jnp.float32),
        mesh=_sc_mesh(),
        scratch_types=[
            pltpu.VMEM((ec, CHUNK), jnp.int32),
            pltpu.VMEM((1, CHUNK), jnp.float32),
            pltpu.VMEM((1, n_pad // NS), jnp.float32),
            pltpu.VMEM_SHARED((n_pad,), jnp.float32),
        ],
    )
    return kern(dst_p)


def _edge_pipeline(z_hbm, src_hbm, dst_hbm, agg_sh, src_v, dst_v, bufs,
                   base, ec):
    hc = ec // 4
    for half in range(4):
        hbase = base + half * hc
        pltpu.sync_copy(src_hbm.at[pl.ds(hbase, hc)], src_v.at[pl.ds(0, hc)])
        pltpu.sync_copy(dst_hbm.at[pl.ds(hbase, hc)], dst_v.at[pl.ds(0, hc)])

        for b, (rows, sem) in enumerate(bufs):
            pltpu.async_copy(z_hbm.at[src_v.at[b]], rows, sem)

        def outer(g, _):
            for b, (rows, sem) in enumerate(bufs):
                j = 2 * g + b
                pltpu.make_async_copy(z_hbm.at[src_v.at[j]], rows, sem).wait()
                pltpu.sync_copy(rows, agg_sh.at[dst_v.at[j]], add=True)

                @pl.when(j + 2 < hc)
                def _():
                    pltpu.async_copy(z_hbm.at[src_v.at[j + 2]], rows, sem)
            return _
        lax.fori_loop(0, hc // 2, outer, None)


def _agg_body(ec_f, ec_s, n_pad, z_hbm, src_hbm, dst_hbm, agg_hbm,
              src_v, dst_v, rows0, rows1, agg_sh, sem0, sem1):
    c = lax.axis_index("c")
    s = lax.axis_index("s")
    rpt = n_pad // NS
    bufs = ((rows0, sem0), (rows1, sem1))

    def work(base, ec):
        _fill_f32(rows0, CHUNK, 128, 0.0)

        def zero(i, _):
            pltpu.sync_copy(rows0,
                            agg_sh.at[pl.ds(s * rpt + i * CHUNK, CHUNK)])
            return _
        lax.fori_loop(0, rpt // CHUNK, zero, None)
        plsc.subcore_barrier()

        _edge_pipeline(z_hbm, src_hbm, dst_hbm, agg_sh, src_v, dst_v, bufs,
                       base, ec)
        plsc.subcore_barrier()

        def out(i, _):
            sl = pl.ds(s * rpt + i * CHUNK, CHUNK)
            pltpu.sync_copy(agg_sh.at[sl], rows0)
            pltpu.sync_copy(rows0, agg_hbm.at[c, sl])
            return _
        lax.fori_loop(0, rpt // CHUNK, out, None)

    if ec_s == 0:
        @pl.when(c != SLOW_C)
        def _():
            work(s * ec_f, ec_f)
    else:
        @pl.when(c == SLOW_C)
        def _():
            work(NS * ec_f + s * ec_s, ec_s)

        @pl.when(c != SLOW_C)
        def _():
            work(s * ec_f, ec_f)


def _sc_aggregate(z, src_p, dst_p, n_pad):
    nchunks = src_p.shape[0]
    per_pair = nchunks // NS
    ec_f = min(per_pair, max(16, round(per_pair * FAST_FRAC / 16) * 16))
    ec_s = per_pair - ec_f
    kern = pl.kernel(
        functools.partial(_agg_body, ec_f, ec_s, n_pad),
        out_type=jax.ShapeDtypeStruct((NC, n_pad, 128), jnp.float32),
        mesh=_sc_mesh(),
        scratch_types=[
            pltpu.VMEM((max(ec_f, ec_s) // 4, CHUNK), jnp.int32),
            pltpu.VMEM((max(ec_f, ec_s) // 4, CHUNK), jnp.int32),
            pltpu.VMEM((CHUNK, 128), jnp.float32),
            pltpu.VMEM((CHUNK, 128), jnp.float32),
            pltpu.VMEM_SHARED((n_pad, 128), jnp.float32),
            pltpu.SemaphoreType.DMA,
            pltpu.SemaphoreType.DMA,
        ],
    )
    return kern(z, src_p, dst_p)


def _z_kernel(x_ref, w_ref, deg_ref, z_ref):
    i = pl.program_id(0)
    rb = x_ref.shape[0]
    dv = deg_ref[0, pl.ds(i * rb, rb)] + deg_ref[1, pl.ds(i * rb, rb)] + 1.0
    dinv = lax.rsqrt(dv)
    xw = jnp.dot(x_ref[...], w_ref[...], preferred_element_type=jnp.float32)
    z_ref[...] = xw * dinv[:, None]


def _tc_z(x_pad, w_cat, deg2, rb):
    n_pad, d_in = x_pad.shape
    grid = n_pad // rb
    return pl.pallas_call(
        _z_kernel,
        grid=(grid,),
        in_specs=[
            pl.BlockSpec((rb, d_in), lambda i: (i, 0)),
            pl.BlockSpec((d_in, 128), lambda i: (0, 0)),
            pl.BlockSpec((NC, n_pad), lambda i: (0, 0)),
        ],
        out_specs=pl.BlockSpec((rb, 128), lambda i: (i, 0)),
        out_shape=jax.ShapeDtypeStruct((n_pad, 128), jnp.float32),
    )(x_pad, w_cat, deg2)


def _out_kernel(agg_ref, z_ref, deg_ref, b_ref, out_ref):
    dv = deg_ref[:, 0] + deg_ref[:, 1] + 1.0
    dinv = lax.rsqrt(dv)
    ssum = agg_ref[0] + z_ref[...]
    if agg_ref.shape[0] == 2:
        ssum = ssum + agg_ref[1]
    out_ref[...] = ssum * dinv[:, None] + b_ref[...]


def _tc_out(agg2, z, deg2, b_cat, n, rb, single):
    grid = n // rb
    fc = 1 - SLOW_C
    agg_spec = (pl.BlockSpec((1, rb, 128), lambda i: (fc, i, 0)) if single
                else pl.BlockSpec((NC, rb, 128), lambda i: (0, i, 0)))
    return pl.pallas_call(
        _out_kernel,
        grid=(grid,),
        in_specs=[
            agg_spec,
            pl.BlockSpec((rb, 128), lambda i: (i, 0)),
            pl.BlockSpec((rb, NC), lambda i: (i, 0)),
            pl.BlockSpec((1, 128), lambda i: (0, 0)),
        ],
        out_specs=pl.BlockSpec((rb, 128), lambda i: (i, 0)),
        out_shape=jax.ShapeDtypeStruct((n, 128), jnp.float32),
    )(agg2, z, deg2.T, b_cat)


def kernel(x, edge_index, W_mu, b_mu, W_logstd, b_logstd):
    n, d_in = x.shape
    d_out = W_mu.shape[1]
    e = edge_index.shape[1]

    n_pad = ((n + 1 + NS * 64 - 1) // (NS * 64)) * (NS * 64)
    ec = (e + NW * CHUNK - 1) // (NW * CHUNK)
    ec = ((ec + 15) // 16) * 16
    e_pad = ec * NW * CHUNK

    src = edge_index[0].astype(jnp.int32)
    dst = edge_index[1].astype(jnp.int32)
    pad = jnp.full((e_pad - e,), n, dtype=jnp.int32)
    src_p = jnp.concatenate([src, pad]).reshape(NW * ec, CHUNK)
    dst_p = jnp.concatenate([dst, pad]).reshape(NW * ec, CHUNK)

    x_pad = jnp.concatenate(
        [x, jnp.zeros((n_pad - n, d_in), dtype=x.dtype)], axis=0)
    w_cat = jnp.concatenate([W_mu, W_logstd], axis=1)
    b_cat = jnp.concatenate([b_mu, b_logstd]).reshape(1, 2 * d_out)

    rb = n_pad // 16

    rb_out = max(d for d in range(8, 513, 8) if n % d == 0)

    deg2 = _sc_degree(dst_p, n_pad).reshape(NC, n_pad)
    z = _tc_z(x_pad, w_cat, deg2, rb)
    agg2 = _sc_aggregate(z, src_p, dst_p, n_pad)
    out_full = _tc_out(agg2, z, deg2, b_cat, n, rb_out,
                       single=(FAST_FRAC >= 1.0))
    return (out_full[:, :d_out], out_full[:, d_out:])


# speedup vs baseline: 1.2992x; 1.2992x over previous
import functools

import jax
import jax.numpy as jnp
from jax import lax
from jax.experimental import pallas as pl
from jax.experimental.pallas import tpu as pltpu
from jax.experimental.pallas import tpu_sc as plsc

NC = 2
NS = 16
NW = NC * NS
CHUNK = 64
SLOW_C = 0
FAST_FRAC = 0.9


def _sc_mesh():
    return plsc.VectorSubcoreMesh(
        core_axis_name="c", subcore_axis_name="s", num_cores=NC,
        num_subcores=NS)


def _fill_f32(ref, n_rows, n_cols, value):
    def row(j, _):
        def col(k, _):
            ref[j, pl.ds(k * 16, 16)] = jnp.full((16,), value, jnp.float32)
            return _
        return lax.fori_loop(0, n_cols // 16, col, _)
    lax.fori_loop(0, n_rows, row, None)


def _deg_body(ec, n_pad, dst_hbm, deg_hbm, dst_v, ones_v, zrow_v, deg_sh):
    c = lax.axis_index("c")
    s = lax.axis_index("s")
    wid = s * NC + c
    rpt = n_pad // NS

    _fill_f32(ones_v, 1, CHUNK, 1.0)
    _fill_f32(zrow_v, 1, rpt, 0.0)
    pltpu.sync_copy(zrow_v.at[0], deg_sh.at[pl.ds(s * rpt, rpt)])
    plsc.subcore_barrier()

    pltpu.sync_copy(dst_hbm.at[pl.ds(wid * ec, ec)], dst_v)

    def edge(j, _):
        pltpu.sync_copy(ones_v.at[0], deg_sh.at[dst_v.at[j]], add=True)
        return _
    lax.fori_loop(0, ec, edge, None)
    plsc.subcore_barrier()

    pltpu.sync_copy(deg_sh.at[pl.ds(s * rpt, rpt)],
                    deg_hbm.at[pl.ds(c * n_pad + s * rpt, rpt)])


def _sc_degree(dst_p, n_pad):
    ec = dst_p.shape[0] // NW
    kern = pl.kernel(
        functools.partial(_deg_body, ec, n_pad),
        out_type=jax.ShapeDtypeStruct((NC * n_pad,), jnp.float32),
        mesh=_sc_mesh(),
        scratch_types=[
            pltpu.VMEM((ec, CHUNK), jnp.int32),
            pltpu.VMEM((1, CHUNK), jnp.float32),
            pltpu.VMEM((1, n_pad // NS), jnp.float32),
            pltpu.VMEM_SHARED((n_pad,), jnp.float32),
        ],
    )
    return kern(dst_p)


def _edge_pipeline(z_hbm, src_hbm, dst_hbm, agg_sh, src_v, dst_v, bufs,
                   base, ec):
    hc = ec // 4
    for half in range(4):
        hbase = base + half * hc
        pltpu.sync_copy(src_hbm.at[pl.ds(hbase, hc)], src_v.at[pl.ds(0, hc)])
        pltpu.sync_copy(dst_hbm.at[pl.ds(hbase, hc)], dst_v.at[pl.ds(0, hc)])

        for b, (rows, sem) in enumerate(bufs):
            pltpu.async_copy(z_hbm.at[src_v.at[b]], rows, sem)

        def outer(g, _):
            for b, (rows, sem) in enumerate(bufs):
                j = 2 * g + b
                pltpu.make_async_copy(z_hbm.at[src_v.at[j]], rows, sem).wait()
                pltpu.sync_copy(rows, agg_sh.at[dst_v.at[j]], add=True)

                @pl.when(j + 2 < hc)
                def _():
                    pltpu.async_copy(z_hbm.at[src_v.at[j + 2]], rows, sem)
            return _
        lax.fori_loop(0, hc // 2, outer, None)


def _agg_body(ec_f, ec_s, n_pad, z_hbm, src_hbm, dst_hbm, agg_hbm,
              src_v, dst_v, rows0, rows1, agg_sh, sem0, sem1):
    c = lax.axis_index("c")
    s = lax.axis_index("s")
    rpt = n_pad // NS
    bufs = ((rows0, sem0), (rows1, sem1))

    def work(base, ec):
        _fill_f32(rows0, CHUNK, 128, 0.0)

        def zero(i, _):
            pltpu.async_copy(
                rows0, agg_sh.at[pl.ds(s * rpt + i * CHUNK, CHUNK)], sem0)
            return _
        lax.fori_loop(0, rpt // CHUNK, zero, None)

        def zdrain(i, _):
            pltpu.make_async_copy(
                rows0, agg_sh.at[pl.ds(s * rpt, CHUNK)], sem0).wait()
            return _
        lax.fori_loop(0, rpt // CHUNK, zdrain, None)
        plsc.subcore_barrier()

        _edge_pipeline(z_hbm, src_hbm, dst_hbm, agg_sh, src_v, dst_v, bufs,
                       base, ec)
        plsc.subcore_barrier()

        def out(g, _):
            for b, (rows, sem) in enumerate(bufs):
                ii = 2 * g + b
                sl = pl.ds(s * rpt + ii * CHUNK, CHUNK)

                @pl.when(ii >= 2)
                def _():
                    pltpu.make_async_copy(
                        rows, agg_hbm.at[c, pl.ds(s * rpt, CHUNK)],
                        sem).wait()
                pltpu.sync_copy(agg_sh.at[sl], rows)
                pltpu.async_copy(rows, agg_hbm.at[c, sl], sem)
            return _
        lax.fori_loop(0, rpt // CHUNK // 2, out, None)
        for b, (rows, sem) in enumerate(bufs):
            pltpu.make_async_copy(
                rows, agg_hbm.at[c, pl.ds(s * rpt, CHUNK)], sem).wait()

    if ec_s == 0:
        @pl.when(c != SLOW_C)
        def _():
            work(s * ec_f, ec_f)
    else:
        @pl.when(c == SLOW_C)
        def _():
            work(NS * ec_f + s * ec_s, ec_s)

        @pl.when(c != SLOW_C)
        def _():
            work(s * ec_f, ec_f)


def _sc_aggregate(z, src_p, dst_p, n_pad):
    nchunks = src_p.shape[0]
    per_pair = nchunks // NS
    ec_f = min(per_pair, max(16, round(per_pair * FAST_FRAC / 16) * 16))
    ec_s = per_pair - ec_f
    kern = pl.kernel(
        functools.partial(_agg_body, ec_f, ec_s, n_pad),
        out_type=jax.ShapeDtypeStruct((NC, n_pad, 128), jnp.float32),
        mesh=_sc_mesh(),
        scratch_types=[
            pltpu.VMEM((max(ec_f, ec_s) // 4, CHUNK), jnp.int32),
            pltpu.VMEM((max(ec_f, ec_s) // 4, CHUNK), jnp.int32),
            pltpu.VMEM((CHUNK, 128), jnp.float32),
            pltpu.VMEM((CHUNK, 128), jnp.float32),
            pltpu.VMEM_SHARED((n_pad, 128), jnp.float32),
            pltpu.SemaphoreType.DMA,
            pltpu.SemaphoreType.DMA,
        ],
    )
    return kern(z, src_p, dst_p)


def _z_kernel(x_ref, w_ref, deg_ref, z_ref):
    i = pl.program_id(0)
    rb = x_ref.shape[0]
    dv = deg_ref[0, pl.ds(i * rb, rb)] + deg_ref[1, pl.ds(i * rb, rb)] + 1.0
    dinv = lax.rsqrt(dv)
    xw = jnp.dot(x_ref[...], w_ref[...], preferred_element_type=jnp.float32)
    z_ref[...] = xw * dinv[:, None]


def _tc_z(x_pad, w_cat, deg2, rb):
    n_pad, d_in = x_pad.shape
    grid = n_pad // rb
    return pl.pallas_call(
        _z_kernel,
        grid=(grid,),
        in_specs=[
            pl.BlockSpec((rb, d_in), lambda i: (i, 0)),
            pl.BlockSpec((d_in, 128), lambda i: (0, 0)),
            pl.BlockSpec((NC, n_pad), lambda i: (0, 0)),
        ],
        out_specs=pl.BlockSpec((rb, 128), lambda i: (i, 0)),
        out_shape=jax.ShapeDtypeStruct((n_pad, 128), jnp.float32),
    )(x_pad, w_cat, deg2)


def _out_kernel(agg_ref, z_ref, deg_ref, b_ref, out_ref):
    dv = deg_ref[:, 0] + deg_ref[:, 1] + 1.0
    dinv = lax.rsqrt(dv)
    ssum = agg_ref[0] + z_ref[...]
    if agg_ref.shape[0] == 2:
        ssum = ssum + agg_ref[1]
    out_ref[...] = ssum * dinv[:, None] + b_ref[...]


def _tc_out(agg2, z, deg2, b_cat, n, rb, single):
    grid = n // rb
    fc = 1 - SLOW_C
    agg_spec = (pl.BlockSpec((1, rb, 128), lambda i: (fc, i, 0)) if single
                else pl.BlockSpec((NC, rb, 128), lambda i: (0, i, 0)))
    return pl.pallas_call(
        _out_kernel,
        grid=(grid,),
        in_specs=[
            agg_spec,
            pl.BlockSpec((rb, 128), lambda i: (i, 0)),
            pl.BlockSpec((rb, NC), lambda i: (i, 0)),
            pl.BlockSpec((1, 128), lambda i: (0, 0)),
        ],
        out_specs=pl.BlockSpec((rb, 128), lambda i: (i, 0)),
        out_shape=jax.ShapeDtypeStruct((n, 128), jnp.float32),
    )(agg2, z, deg2.T, b_cat)


def kernel(x, edge_index, W_mu, b_mu, W_logstd, b_logstd):
    n, d_in = x.shape
    d_out = W_mu.shape[1]
    e = edge_index.shape[1]

    n_pad = ((n + 1 + NS * 64 - 1) // (NS * 64)) * (NS * 64)
    ec = (e + NW * CHUNK - 1) // (NW * CHUNK)
    ec = ((ec + 15) // 16) * 16
    e_pad = ec * NW * CHUNK

    src = edge_index[0].astype(jnp.int32)
    dst = edge_index[1].astype(jnp.int32)
    pad = jnp.full((e_pad - e,), n, dtype=jnp.int32)
    src_p = jnp.concatenate([src, pad]).reshape(NW * ec, CHUNK)
    dst_p = jnp.concatenate([dst, pad]).reshape(NW * ec, CHUNK)

    x_pad = jnp.concatenate(
        [x, jnp.zeros((n_pad - n, d_in), dtype=x.dtype)], axis=0)
    w_cat = jnp.concatenate([W_mu, W_logstd], axis=1)
    b_cat = jnp.concatenate([b_mu, b_logstd]).reshape(1, 2 * d_out)

    rb = n_pad // 16

    rb_out = max(d for d in range(8, 513, 8) if n % d == 0)

    deg2 = _sc_degree(dst_p, n_pad).reshape(NC, n_pad)
    z = _tc_z(x_pad, w_cat, deg2, rb)
    agg2 = _sc_aggregate(z, src_p, dst_p, n_pad)
    out_full = _tc_out(agg2, z, deg2, b_cat, n, rb_out,
                       single=(FAST_FRAC >= 1.0))
    return (out_full[:, :d_out], out_full[:, d_out:])


# speedup vs baseline: 1.3010x; 1.0014x over previous
import functools

import jax
import jax.numpy as jnp
from jax import lax
from jax.experimental import pallas as pl
from jax.experimental.pallas import tpu as pltpu
from jax.experimental.pallas import tpu_sc as plsc

NC = 2
NS = 16
NW = NC * NS
CHUNK = 64
SLOW_C = 0
FAST_FRAC = 0.9


def _sc_mesh():
    return plsc.VectorSubcoreMesh(
        core_axis_name="c", subcore_axis_name="s", num_cores=NC,
        num_subcores=NS)


def _fill_f32(ref, n_rows, n_cols, value):
    def row(j, _):
        def col(k, _):
            ref[j, pl.ds(k * 16, 16)] = jnp.full((16,), value, jnp.float32)
            return _
        return lax.fori_loop(0, n_cols // 16, col, _)
    lax.fori_loop(0, n_rows, row, None)


def _deg_body(ec, n_pad, dst_hbm, deg_hbm, dst_v, ones_v, zrow_v, deg_sh):
    c = lax.axis_index("c")
    s = lax.axis_index("s")
    wid = s * NC + c
    rpt = n_pad // NS

    _fill_f32(ones_v, 1, CHUNK, 1.0)
    _fill_f32(zrow_v, 1, rpt, 0.0)
    pltpu.sync_copy(zrow_v.at[0], deg_sh.at[pl.ds(s * rpt, rpt)])
    plsc.subcore_barrier()

    pltpu.sync_copy(dst_hbm.at[pl.ds(wid * ec, ec)], dst_v)

    def edge(j, _):
        pltpu.sync_copy(ones_v.at[0], deg_sh.at[dst_v.at[j]], add=True)
        return _
    lax.fori_loop(0, ec, edge, None)
    plsc.subcore_barrier()

    pltpu.sync_copy(deg_sh.at[pl.ds(s * rpt, rpt)],
                    deg_hbm.at[pl.ds(c * n_pad + s * rpt, rpt)])


def _sc_degree(dst_p, n_pad):
    ec = dst_p.shape[0] // NW
    kern = pl.kernel(
        functools.partial(_deg_body, ec, n_pad),
        out_type=jax.ShapeDtypeStruct((NC * n_pad,), jnp.float32),
        mesh=_sc_mesh(),
        scratch_types=[
            pltpu.VMEM((ec, CHUNK), jnp.int32),
            pltpu.VMEM((1, CHUNK), jnp.float32),
            pltpu.VMEM((1, n_pad // NS), jnp.float32),
            pltpu.VMEM_SHARED((n_pad,), jnp.float32),
        ],
    )
    return kern(dst_p)


def _edge_pipeline(z_hbm, src_hbm, dst_hbm, agg_sh, src_v, dst_v, bufs,
                   base, ec):
    nstg = 4 if ec >= 128 else 1
    hc = ec // nstg
    for half in range(nstg):
        hbase = base + half * hc
        pltpu.sync_copy(src_hbm.at[pl.ds(hbase, hc)], src_v.at[pl.ds(0, hc)])
        pltpu.sync_copy(dst_hbm.at[pl.ds(hbase, hc)], dst_v.at[pl.ds(0, hc)])

        for b, (rows, sem) in enumerate(bufs):
            pltpu.async_copy(z_hbm.at[src_v.at[b]], rows, sem)

        def outer(g, _):
            for b, (rows, sem) in enumerate(bufs):
                j = 2 * g + b
                pltpu.make_async_copy(z_hbm.at[src_v.at[j]], rows, sem).wait()
                pltpu.sync_copy(rows, agg_sh.at[dst_v.at[j]], add=True)

                @pl.when(j + 2 < hc)
                def _():
                    pltpu.async_copy(z_hbm.at[src_v.at[j + 2]], rows, sem)
            return _
        lax.fori_loop(0, hc // 2, outer, None)


def _agg_body(ec_f, ec_s, n_pad, z_hbm, src_hbm, dst_hbm, agg_hbm,
              src_v, dst_v, rows0, rows1, agg_sh, sem0, sem1):
    c = lax.axis_index("c")
    s = lax.axis_index("s")
    rpt = n_pad // NS
    bufs = ((rows0, sem0), (rows1, sem1))

    def work(base, ec):
        _fill_f32(rows0, CHUNK, 128, 0.0)

        def zero(i, _):
            pltpu.async_copy(
                rows0, agg_sh.at[pl.ds(s * rpt + i * CHUNK, CHUNK)], sem0)
            return _
        lax.fori_loop(0, rpt // CHUNK, zero, None)

        def zdrain(i, _):
            pltpu.make_async_copy(
                rows0, agg_sh.at[pl.ds(s * rpt, CHUNK)], sem0).wait()
            return _
        lax.fori_loop(0, rpt // CHUNK, zdrain, None)
        plsc.subcore_barrier()

        _edge_pipeline(z_hbm, src_hbm, dst_hbm, agg_sh, src_v, dst_v, bufs,
                       base, ec)
        plsc.subcore_barrier()

        def out(g, _):
            for b, (rows, sem) in enumerate(bufs):
                ii = 2 * g + b
                sl = pl.ds(s * rpt + ii * CHUNK, CHUNK)

                @pl.when(ii >= 2)
                def _():
                    pltpu.make_async_copy(
                        rows, agg_hbm.at[c, pl.ds(s * rpt, CHUNK)],
                        sem).wait()
                pltpu.sync_copy(agg_sh.at[sl], rows)
                pltpu.async_copy(rows, agg_hbm.at[c, sl], sem)
            return _
        lax.fori_loop(0, rpt // CHUNK // 2, out, None)
        for b, (rows, sem) in enumerate(bufs):
            pltpu.make_async_copy(
                rows, agg_hbm.at[c, pl.ds(s * rpt, CHUNK)], sem).wait()

    if ec_s == 0:
        @pl.when(c != SLOW_C)
        def _():
            work(s * ec_f, ec_f)
    else:
        @pl.when(c == SLOW_C)
        def _():
            work(NS * ec_f + s * ec_s, ec_s)

        @pl.when(c != SLOW_C)
        def _():
            work(s * ec_f, ec_f)


def _sc_aggregate(z, src_p, dst_p, n_pad):
    nchunks = src_p.shape[0]
    per_pair = nchunks // NS
    ec_f = min(per_pair, max(16, round(per_pair * FAST_FRAC / 16) * 16))
    ec_s = per_pair - ec_f
    ixrows = max(ec // (4 if ec >= 128 else 1) for ec in (ec_f, max(ec_s, 1)))
    kern = pl.kernel(
        functools.partial(_agg_body, ec_f, ec_s, n_pad),
        out_type=jax.ShapeDtypeStruct((NC, n_pad, 128), jnp.float32),
        mesh=_sc_mesh(),
        scratch_types=[
            pltpu.VMEM((ixrows, CHUNK), jnp.int32),
            pltpu.VMEM((ixrows, CHUNK), jnp.int32),
            pltpu.VMEM((CHUNK, 128), jnp.float32),
            pltpu.VMEM((CHUNK, 128), jnp.float32),
            pltpu.VMEM_SHARED((n_pad, 128), jnp.float32),
            pltpu.SemaphoreType.DMA,
            pltpu.SemaphoreType.DMA,
        ],
    )
    return kern(z, src_p, dst_p)


def _z_kernel(x_ref, w_ref, deg_ref, z_ref):
    i = pl.program_id(0)
    rb = x_ref.shape[0]
    dv = deg_ref[0, pl.ds(i * rb, rb)] + deg_ref[1, pl.ds(i * rb, rb)] + 1.0
    dinv = lax.rsqrt(dv)
    xw = jnp.dot(x_ref[...], w_ref[...], preferred_element_type=jnp.float32)
    z_ref[...] = xw * dinv[:, None]


def _tc_z(x_pad, w_cat, deg2, rb):
    n_pad, d_in = x_pad.shape
    grid = n_pad // rb
    return pl.pallas_call(
        _z_kernel,
        grid=(grid,),
        in_specs=[
            pl.BlockSpec((rb, d_in), lambda i: (i, 0)),
            pl.BlockSpec((d_in, 128), lambda i: (0, 0)),
            pl.BlockSpec((NC, n_pad), lambda i: (0, 0)),
        ],
        out_specs=pl.BlockSpec((rb, 128), lambda i: (i, 0)),
        out_shape=jax.ShapeDtypeStruct((n_pad, 128), jnp.float32),
    )(x_pad, w_cat, deg2)


def _out_kernel(agg_ref, z_ref, deg_ref, b_ref, out_ref):
    dv = deg_ref[:, 0] + deg_ref[:, 1] + 1.0
    dinv = lax.rsqrt(dv)
    ssum = agg_ref[0] + z_ref[...]
    if agg_ref.shape[0] == 2:
        ssum = ssum + agg_ref[1]
    out_ref[...] = ssum * dinv[:, None] + b_ref[...]


def _tc_out(agg2, z, deg2, b_cat, n, rb, single):
    grid = n // rb
    fc = 1 - SLOW_C
    agg_spec = (pl.BlockSpec((1, rb, 128), lambda i: (fc, i, 0)) if single
                else pl.BlockSpec((NC, rb, 128), lambda i: (0, i, 0)))
    return pl.pallas_call(
        _out_kernel,
        grid=(grid,),
        in_specs=[
            agg_spec,
            pl.BlockSpec((rb, 128), lambda i: (i, 0)),
            pl.BlockSpec((rb, NC), lambda i: (i, 0)),
            pl.BlockSpec((1, 128), lambda i: (0, 0)),
        ],
        out_specs=pl.BlockSpec((rb, 128), lambda i: (i, 0)),
        out_shape=jax.ShapeDtypeStruct((n, 128), jnp.float32),
    )(agg2, z, deg2.T, b_cat)


def kernel(x, edge_index, W_mu, b_mu, W_logstd, b_logstd):
    n, d_in = x.shape
    d_out = W_mu.shape[1]
    e = edge_index.shape[1]

    n_pad = ((n + 1 + NS * 64 - 1) // (NS * 64)) * (NS * 64)
    ec = (e + NW * CHUNK - 1) // (NW * CHUNK)
    ec = ((ec + 15) // 16) * 16
    e_pad = ec * NW * CHUNK

    src = edge_index[0].astype(jnp.int32)
    dst = edge_index[1].astype(jnp.int32)
    pad = jnp.full((e_pad - e,), n, dtype=jnp.int32)
    src_p = jnp.concatenate([src, pad]).reshape(NW * ec, CHUNK)
    dst_p = jnp.concatenate([dst, pad]).reshape(NW * ec, CHUNK)

    x_pad = jnp.concatenate(
        [x, jnp.zeros((n_pad - n, d_in), dtype=x.dtype)], axis=0)
    w_cat = jnp.concatenate([W_mu, W_logstd], axis=1)
    b_cat = jnp.concatenate([b_mu, b_logstd]).reshape(1, 2 * d_out)

    rb = n_pad // 16

    rb_out = max(d for d in range(8, 513, 8) if n % d == 0)

    deg2 = _sc_degree(dst_p, n_pad).reshape(NC, n_pad)
    z = _tc_z(x_pad, w_cat, deg2, rb)
    agg2 = _sc_aggregate(z, src_p, dst_p, n_pad)
    out_full = _tc_out(agg2, z, deg2, b_cat, n, rb_out,
                       single=(FAST_FRAC >= 1.0))
    return (out_full[:, :d_out], out_full[:, d_out:])
